# baseline (speedup 1.0000x reference)
import functools

import jax
import jax.numpy as jnp
from jax import lax
from jax.experimental import pallas as pl
from jax.experimental.pallas import tpu as pltpu
from jax.experimental.pallas import tpu_sc as plsc

D = 64
DP = 128
NC = 2
NS = 16
NW = NC * NS
LB = 128


@functools.cache
def _fmt_fn(V: int):
  nblk = V // LB
  rem = V - nblk * LB
  pw = (nblk + NW - 1) // NW
  extra = nblk - (pw - 1) * NW
  mesh = plsc.VectorSubcoreMesh(core_axis_name="c", subcore_axis_name="s")

  @functools.partial(
      pl.kernel,
      out_type=jax.ShapeDtypeStruct((V // 2, DP), jnp.float32),
      mesh=mesh,
      compiler_params=pltpu.CompilerParams(
          use_tc_tiling_on_sc=True, needs_layout_passes=False),
      scratch_types=[
          pltpu.VMEM((2, D, LB), jnp.float32),
          pltpu.VMEM((2, LB // 2, DP), jnp.float32),
          pltpu.VMEM((D, LB // 2), jnp.float32),
          pltpu.SemaphoreType.DMA,
          pltpu.SemaphoreType.DMA,
      ],
  )
  def fmt(xt_hbm, t_hbm, ibuf, obuf, tbuf, isem, osem):
    wid = lax.axis_index("s") * NC + lax.axis_index("c")
    cnt = jnp.where(wid < extra, pw, pw - 1)

    def off(k):
      return (wid + k * NW) * LB

    def idesc(k, slot):
      return pltpu.make_async_copy(
          xt_hbm.at[:, pl.ds(off(k), LB)], ibuf.at[slot], isem)

    def odesc(k, slot):
      return pltpu.make_async_copy(
          obuf.at[slot],
          t_hbm.at[pl.ds((wid + k * NW) * (LB // 2), LB // 2)], osem)

    iota16 = lax.broadcasted_iota(jnp.int32, (16,), 0)
    rots = [(iota16 + r) & 15 for r in range(16)]
    hiota = iota16 >> 1
    pcs = [((iota16 & 1) << 6) + rv for rv in rots]

    idesc(0, 0).start()

    def body(k, carry):
      slot = k % 2
      valid = k < cnt

      @pl.when(valid)
      def _():
        idesc(k, slot).wait()

        @pl.when(k + 1 < cnt)
        def _():
          idesc(k + 1, 1 - slot).start()

        @pl.when(k >= 2)
        def _():
          odesc(k - 2, slot).wait()

      @pl.when(valid)
      def _():
        def tp(bb, c):
          rvec = iota16 + bb * 16
          pvec = hiota + bb * 8
          for dg in range(D // 16):
            dvecs = [rots[r] + dg * 16 for r in range(16)]
            cvecs = [pc + dg * 16 for pc in pcs]
            vs = [plsc.load_gather(ibuf.at[slot], [dv, rvec]) for dv in dvecs]
            for cv, v in zip(cvecs, vs):
              plsc.store_scatter(obuf.at[slot], [pvec, cv], v)
          return c

        lax.fori_loop(0, LB // 16, tp, 0)
        odesc(k, slot).start()

      return carry

    lax.fori_loop(0, pw, body, 0)
    odesc(cnt - 2, (cnt - 2) % 2).wait()
    odesc(cnt - 1, (cnt - 1) % 2).wait()

    if rem:
      @pl.when(wid == extra)
      def _():
        pltpu.sync_copy(xt_hbm.at[:, pl.ds(nblk * LB, rem)], tbuf)

        def ttp(bb, c):
          rvec = iota16 + bb * 16
          pvec = hiota + bb * 8
          for dg in range(D // 16):
            dvecs = [rots[r] + dg * 16 for r in range(16)]
            cvecs = [pc + dg * 16 for pc in pcs]
            vs = [plsc.load_gather(tbuf, [dv, rvec]) for dv in dvecs]
            for cv, v in zip(cvecs, vs):
              plsc.store_scatter(obuf.at[0], [pvec, cv], v)
          return c

        lax.fori_loop(0, rem // 16, ttp, 0)
        pltpu.sync_copy(
            obuf.at[0, 0:rem // 2],
            t_hbm.at[pl.ds(nblk * LB // 2, rem // 2)])

  return fmt


@functools.cache
def _gather_fn(B: int, H: int):
  n_slabs = H * (B // LB)
  per_w = n_slabs // NW
  jb_per_h = B // LB
  mesh = plsc.VectorSubcoreMesh(core_axis_name="c", subcore_axis_name="s")

  @functools.partial(
      pl.kernel,
      out_type=jax.ShapeDtypeStruct((H, D, B), jnp.float32),
      mesh=mesh,
      compiler_params=pltpu.CompilerParams(
          use_tc_tiling_on_sc=True, needs_layout_passes=False),
      scratch_types=[
          pltpu.VMEM((2, LB), jnp.int32),
          pltpu.VMEM((2, LB), jnp.int32),
          pltpu.VMEM((2, LB, DP), jnp.float32),
          pltpu.VMEM((2, D, LB), jnp.float32),
          pltpu.SemaphoreType.DMA,
          pltpu.SemaphoreType.DMA,
          pltpu.SemaphoreType.DMA,
      ],
  )
  def gather(table_hbm, xt_hbm, y_hbm, idxb, pidxb, rowb, obuf, isem, gsem,
             wsem):
    wid = lax.axis_index("s") * NC + lax.axis_index("c")
    base = wid * per_w

    def slab_hjb(i):
      s = base + i
      return s // jb_per_h, s % jb_per_h

    def idesc(i, slot):
      h, jb = slab_hjb(i)
      return pltpu.make_async_copy(
          xt_hbm.at[h, pl.ds(jb * LB, LB)], idxb.at[slot], isem)

    def gdesc(i, slot):
      return pltpu.make_async_copy(
          table_hbm.at[pidxb.at[slot]], rowb.at[slot], gsem)

    def wdesc(i, slot):
      h, jb = slab_hjb(i)
      return pltpu.make_async_copy(
          obuf.at[slot], y_hbm.at[h, :, pl.ds(jb * LB, LB)], wsem)

    def mkpidx(slot):
      for g in range(LB // 16):
        pidxb[slot, pl.ds(g * 16, 16)] = idxb[slot, pl.ds(g * 16, 16)] >> 1

    idesc(0, 0).start()
    idesc(1, 1).start()
    idesc(0, 0).wait()
    mkpidx(0)
    gdesc(0, 0).start()

    def body(i, carry):
      slot = i % 2
      nxt = 1 - slot

      @pl.when(i + 1 < per_w)
      def _():
        idesc(i + 1, nxt).wait()
        mkpidx(nxt)
        gdesc(i + 1, nxt).start()

      gdesc(i, slot).wait()

      @pl.when(i >= 2)
      def _():
        wdesc(i - 2, slot).wait()

      iota16 = lax.broadcasted_iota(jnp.int32, (16,), 0)
      rots = [(iota16 + r) & 15 for r in range(16)]

      def arrange(bb, c):
        bvec = iota16 + bb * 16
        pv = (idxb[slot, pl.ds(bb * 16, 16)] & 1) << 6
        for dg in range(D // 16):
          dvecs = [rots[r] + dg * 16 for r in range(16)]
          vs = [
              plsc.load_gather(rowb.at[slot], [bvec, dv + pv]) for dv in dvecs
          ]
          for dv, v in zip(dvecs, vs):
            plsc.store_scatter(obuf.at[slot], [dv, bvec], v)
        return c

      lax.fori_loop(0, LB // 16, arrange, 0)

      @pl.when(i + 2 < per_w)
      def _():
        idesc(i + 2, slot).start()

      wdesc(i, slot).start()
      return carry

    lax.fori_loop(0, per_w, body, 0)
    wdesc(per_w - 2, per_w % 2).wait()
    wdesc(per_w - 1, (per_w - 1) % 2).wait()

  return gather


def kernel(x, x_0):
  B, H = x.shape
  V = x_0.shape[0]
  table = _fmt_fn(V)(x_0.T)
  y = _gather_fn(B, H)(table, x.T)
  return jnp.transpose(y, (2, 0, 1))


# speedup vs baseline: 1.0937x; 1.0937x over previous
import functools

import jax
import jax.numpy as jnp
from jax import lax
from jax.experimental import pallas as pl
from jax.experimental.pallas import tpu as pltpu
from jax.experimental.pallas import tpu_sc as plsc

D = 64
DP = 128
NC = 2
NS = 16
NW = NC * NS
LB = 128


@functools.cache
def _fmt_fn(V: int):
  CB = 2 * LB
  nblk = V // CB
  rem = V - nblk * CB
  pw = (nblk + NW - 1) // NW
  extra = nblk - (pw - 1) * NW
  mesh = plsc.VectorSubcoreMesh(core_axis_name="c", subcore_axis_name="s")

  @functools.partial(
      pl.kernel,
      out_type=jax.ShapeDtypeStruct((V // 2, DP), jnp.float32),
      mesh=mesh,
      compiler_params=pltpu.CompilerParams(
          use_tc_tiling_on_sc=True, needs_layout_passes=False),
      scratch_types=[
          pltpu.VMEM((2, D, CB), jnp.float32),
          pltpu.VMEM((2, CB // 2, DP), jnp.float32),
          pltpu.VMEM((D, LB // 2), jnp.float32),
          pltpu.SemaphoreType.DMA,
          pltpu.SemaphoreType.DMA,
      ],
  )
  def fmt(xt_hbm, t_hbm, ibuf, obuf, tbuf, isem, osem):
    wid = lax.axis_index("s") * NC + lax.axis_index("c")
    cnt = jnp.where(wid < extra, pw, pw - 1)

    def off(k):
      return (wid + k * NW) * CB

    def idesc(k, slot):
      return pltpu.make_async_copy(
          xt_hbm.at[:, pl.ds(off(k), CB)], ibuf.at[slot], isem)

    def odesc(k, slot):
      return pltpu.make_async_copy(
          obuf.at[slot],
          t_hbm.at[pl.ds((wid + k * NW) * (CB // 2), CB // 2)], osem)

    iota16 = lax.broadcasted_iota(jnp.int32, (16,), 0)
    rots = [(iota16 + r) & 15 for r in range(16)]
    hiota = iota16 >> 1
    pcs = [((iota16 & 1) << 6) + rv for rv in rots]

    idesc(0, 0).start()

    def body(k, carry):
      slot = k % 2
      valid = k < cnt

      @pl.when(valid)
      def _():
        idesc(k, slot).wait()

        @pl.when(k + 1 < cnt)
        def _():
          idesc(k + 1, 1 - slot).start()

        @pl.when(k >= 2)
        def _():
          odesc(k - 2, slot).wait()

      @pl.when(valid)
      def _():
        def tp(bb, c):
          rvec = iota16 + bb * 16
          pvec = hiota + bb * 8
          for dg in range(D // 16):
            dvecs = [rots[r] + dg * 16 for r in range(16)]
            cvecs = [pc + dg * 16 for pc in pcs]
            vs = [plsc.load_gather(ibuf.at[slot], [dv, rvec]) for dv in dvecs]
            for cv, v in zip(cvecs, vs):
              plsc.store_scatter(obuf.at[slot], [pvec, cv], v)
          return c

        lax.fori_loop(0, CB // 16, tp, 0)
        odesc(k, slot).start()

      return carry

    lax.fori_loop(0, pw, body, 0)
    odesc(cnt - 2, (cnt - 2) % 2).wait()
    odesc(cnt - 1, (cnt - 1) % 2).wait()

    if rem:
      @pl.when(wid == extra)
      def _():
        pltpu.sync_copy(xt_hbm.at[:, pl.ds(nblk * CB, rem)], tbuf)

        def ttp(bb, c):
          rvec = iota16 + bb * 16
          pvec = hiota + bb * 8
          for dg in range(D // 16):
            dvecs = [rots[r] + dg * 16 for r in range(16)]
            cvecs = [pc + dg * 16 for pc in pcs]
            vs = [plsc.load_gather(tbuf, [dv, rvec]) for dv in dvecs]
            for cv, v in zip(cvecs, vs):
              plsc.store_scatter(obuf.at[0], [pvec, cv], v)
          return c

        lax.fori_loop(0, rem // 16, ttp, 0)
        pltpu.sync_copy(
            obuf.at[0, 0:rem // 2],
            t_hbm.at[pl.ds(nblk * (CB // 2), rem // 2)])

  return fmt


@functools.cache
def _gather_fn(B: int, H: int):
  n_slabs = H * (B // LB)
  per_w = n_slabs // NW
  jb_per_h = B // LB
  mesh = plsc.VectorSubcoreMesh(core_axis_name="c", subcore_axis_name="s")

  @functools.partial(
      pl.kernel,
      out_type=jax.ShapeDtypeStruct((H, D, B), jnp.float32),
      mesh=mesh,
      compiler_params=pltpu.CompilerParams(
          use_tc_tiling_on_sc=True, needs_layout_passes=False),
      scratch_types=[
          pltpu.VMEM((2, LB), jnp.int32),
          pltpu.VMEM((2, LB), jnp.int32),
          pltpu.VMEM((2, LB, DP), jnp.float32),
          pltpu.VMEM((2, D, LB), jnp.float32),
          pltpu.SemaphoreType.DMA,
          pltpu.SemaphoreType.DMA,
          pltpu.SemaphoreType.DMA,
      ],
  )
  def gather(table_hbm, xt_hbm, y_hbm, idxb, pidxb, rowb, obuf, isem, gsem,
             wsem):
    wid = lax.axis_index("s") * NC + lax.axis_index("c")
    base = wid * per_w

    def slab_hjb(i):
      s = base + i
      return s // jb_per_h, s % jb_per_h

    def idesc(i, slot):
      h, jb = slab_hjb(i)
      return pltpu.make_async_copy(
          xt_hbm.at[h, pl.ds(jb * LB, LB)], idxb.at[slot], isem)

    def gdesc(i, slot):
      return pltpu.make_async_copy(
          table_hbm.at[pidxb.at[slot]], rowb.at[slot], gsem)

    def wdesc(i, slot):
      h, jb = slab_hjb(i)
      return pltpu.make_async_copy(
          obuf.at[slot], y_hbm.at[h, :, pl.ds(jb * LB, LB)], wsem)

    def mkpidx(slot):
      for g in range(LB // 16):
        pidxb[slot, pl.ds(g * 16, 16)] = idxb[slot, pl.ds(g * 16, 16)] >> 1

    idesc(0, 0).start()
    idesc(1, 1).start()
    idesc(0, 0).wait()
    mkpidx(0)
    gdesc(0, 0).start()

    def body(i, carry):
      slot = i % 2
      nxt = 1 - slot

      @pl.when(i + 1 < per_w)
      def _():
        idesc(i + 1, nxt).wait()
        mkpidx(nxt)
        gdesc(i + 1, nxt).start()

      gdesc(i, slot).wait()

      @pl.when(i >= 2)
      def _():
        wdesc(i - 2, slot).wait()

      iota16 = lax.broadcasted_iota(jnp.int32, (16,), 0)
      rots = [(iota16 + r) & 15 for r in range(16)]

      def arrange(bb, c):
        bvec = iota16 + bb * 16
        pv = (idxb[slot, pl.ds(bb * 16, 16)] & 1) << 6
        for dg in range(D // 16):
          dvecs = [rots[r] + dg * 16 for r in range(16)]
          vs = [
              plsc.load_gather(rowb.at[slot], [bvec, dv + pv]) for dv in dvecs
          ]
          for dv, v in zip(dvecs, vs):
            plsc.store_scatter(obuf.at[slot], [dv, bvec], v)
        return c

      lax.fori_loop(0, LB // 16, arrange, 0)

      @pl.when(i + 2 < per_w)
      def _():
        idesc(i + 2, slot).start()

      wdesc(i, slot).start()
      return carry

    lax.fori_loop(0, per_w, body, 0)
    wdesc(per_w - 2, per_w % 2).wait()
    wdesc(per_w - 1, (per_w - 1) % 2).wait()

  return gather


def kernel(x, x_0):
  B, H = x.shape
  V = x_0.shape[0]
  table = _fmt_fn(V)(x_0.T)
  y = _gather_fn(B, H)(table, x.T)
  return jnp.transpose(y, (2, 0, 1))


# speedup vs baseline: 1.0937x; 1.0000x over previous
import functools

import jax
import jax.numpy as jnp
from jax import lax
from jax.experimental import pallas as pl
from jax.experimental.pallas import tpu as pltpu
from jax.experimental.pallas import tpu_sc as plsc

D = 64
DP = 128
NC = 2
NS = 16
NW = NC * NS
LB = 128


@functools.cache
def _fmt_fn(V: int):
  CB = 3 * LB
  nblk = V // CB
  rem = V - nblk * CB
  pw = (nblk + NW - 1) // NW
  extra = nblk - (pw - 1) * NW
  mesh = plsc.VectorSubcoreMesh(core_axis_name="c", subcore_axis_name="s")

  @functools.partial(
      pl.kernel,
      out_type=jax.ShapeDtypeStruct((V // 2, DP), jnp.float32),
      mesh=mesh,
      compiler_params=pltpu.CompilerParams(
          use_tc_tiling_on_sc=True, needs_layout_passes=False),
      scratch_types=[
          pltpu.VMEM((2, D, CB), jnp.float32),
          pltpu.VMEM((2, CB // 2, DP), jnp.float32),
          pltpu.VMEM((D, LB // 2), jnp.float32),
          pltpu.SemaphoreType.DMA,
          pltpu.SemaphoreType.DMA,
      ],
  )
  def fmt(xt_hbm, t_hbm, ibuf, obuf, tbuf, isem, osem):
    wid = lax.axis_index("s") * NC + lax.axis_index("c")
    cnt = jnp.where(wid < extra, pw, pw - 1)

    def off(k):
      return (wid + k * NW) * CB

    def idesc(k, slot):
      return pltpu.make_async_copy(
          xt_hbm.at[:, pl.ds(off(k), CB)], ibuf.at[slot], isem)

    def odesc(k, slot):
      return pltpu.make_async_copy(
          obuf.at[slot],
          t_hbm.at[pl.ds((wid + k * NW) * (CB // 2), CB // 2)], osem)

    iota16 = lax.broadcasted_iota(jnp.int32, (16,), 0)
    rots = [(iota16 + r) & 15 for r in range(16)]
    hiota = iota16 >> 1
    pcs = [((iota16 & 1) << 6) + rv for rv in rots]

    idesc(0, 0).start()

    def body(k, carry):
      slot = k % 2
      valid = k < cnt

      @pl.when(valid)
      def _():
        idesc(k, slot).wait()

        @pl.when(k + 1 < cnt)
        def _():
          idesc(k + 1, 1 - slot).start()

        @pl.when(k >= 2)
        def _():
          odesc(k - 2, slot).wait()

      @pl.when(valid)
      def _():
        def tp(bb, c):
          rvec = iota16 + bb * 16
          pvec = hiota + bb * 8
          for dg in range(D // 16):
            dvecs = [rots[r] + dg * 16 for r in range(16)]
            cvecs = [pc + dg * 16 for pc in pcs]
            vs = [plsc.load_gather(ibuf.at[slot], [dv, rvec]) for dv in dvecs]
            for cv, v in zip(cvecs, vs):
              plsc.store_scatter(obuf.at[slot], [pvec, cv], v)
          return c

        lax.fori_loop(0, CB // 16, tp, 0)
        odesc(k, slot).start()

      return carry

    lax.fori_loop(0, pw, body, 0)
    odesc(cnt - 2, (cnt - 2) % 2).wait()
    odesc(cnt - 1, (cnt - 1) % 2).wait()

    if rem:
      @pl.when(wid == extra)
      def _():
        pltpu.sync_copy(xt_hbm.at[:, pl.ds(nblk * CB, rem)], tbuf)

        def ttp(bb, c):
          rvec = iota16 + bb * 16
          pvec = hiota + bb * 8
          for dg in range(D // 16):
            dvecs = [rots[r] + dg * 16 for r in range(16)]
            cvecs = [pc + dg * 16 for pc in pcs]
            vs = [plsc.load_gather(tbuf, [dv, rvec]) for dv in dvecs]
            for cv, v in zip(cvecs, vs):
              plsc.store_scatter(obuf.at[0], [pvec, cv], v)
          return c

        lax.fori_loop(0, rem // 16, ttp, 0)
        pltpu.sync_copy(
            obuf.at[0, 0:rem // 2],
            t_hbm.at[pl.ds(nblk * (CB // 2), rem // 2)])

  return fmt


@functools.cache
def _gather_fn(B: int, H: int):
  n_slabs = H * (B // LB)
  per_w = n_slabs // NW
  jb_per_h = B // LB
  mesh = plsc.VectorSubcoreMesh(core_axis_name="c", subcore_axis_name="s")

  @functools.partial(
      pl.kernel,
      out_type=jax.ShapeDtypeStruct((H, D, B), jnp.float32),
      mesh=mesh,
      compiler_params=pltpu.CompilerParams(
          use_tc_tiling_on_sc=True, needs_layout_passes=False),
      scratch_types=[
          pltpu.VMEM((2, LB), jnp.int32),
          pltpu.VMEM((2, LB), jnp.int32),
          pltpu.VMEM((2, LB, DP), jnp.float32),
          pltpu.VMEM((2, D, LB), jnp.float32),
          pltpu.SemaphoreType.DMA,
          pltpu.SemaphoreType.DMA,
          pltpu.SemaphoreType.DMA,
      ],
  )
  def gather(table_hbm, xt_hbm, y_hbm, idxb, pidxb, rowb, obuf, isem, gsem,
             wsem):
    wid = lax.axis_index("s") * NC + lax.axis_index("c")
    base = wid * per_w

    def slab_hjb(i):
      s = base + i
      return s // jb_per_h, s % jb_per_h

    def idesc(i, slot):
      h, jb = slab_hjb(i)
      return pltpu.make_async_copy(
          xt_hbm.at[h, pl.ds(jb * LB, LB)], idxb.at[slot], isem)

    def gdesc(i, slot):
      return pltpu.make_async_copy(
          table_hbm.at[pidxb.at[slot]], rowb.at[slot], gsem)

    def wdesc(i, slot):
      h, jb = slab_hjb(i)
      return pltpu.make_async_copy(
          obuf.at[slot], y_hbm.at[h, :, pl.ds(jb * LB, LB)], wsem)

    def mkpidx(slot):
      for g in range(LB // 16):
        pidxb[slot, pl.ds(g * 16, 16)] = idxb[slot, pl.ds(g * 16, 16)] >> 1

    idesc(0, 0).start()
    idesc(1, 1).start()
    idesc(0, 0).wait()
    mkpidx(0)
    gdesc(0, 0).start()

    def body(i, carry):
      slot = i % 2
      nxt = 1 - slot

      @pl.when(i + 1 < per_w)
      def _():
        idesc(i + 1, nxt).wait()
        mkpidx(nxt)
        gdesc(i + 1, nxt).start()

      gdesc(i, slot).wait()

      @pl.when(i >= 2)
      def _():
        wdesc(i - 2, slot).wait()

      iota16 = lax.broadcasted_iota(jnp.int32, (16,), 0)
      rots = [(iota16 + r) & 15 for r in range(16)]

      def arrange(bb, c):
        bvec = iota16 + bb * 16
        pv = (idxb[slot, pl.ds(bb * 16, 16)] & 1) << 6
        for dg in range(D // 16):
          dvecs = [rots[r] + dg * 16 for r in range(16)]
          vs = [
              plsc.load_gather(rowb.at[slot], [bvec, dv + pv]) for dv in dvecs
          ]
          for dv, v in zip(dvecs, vs):
            plsc.store_scatter(obuf.at[slot], [dv, bvec], v)
        return c

      lax.fori_loop(0, LB // 16, arrange, 0)

      @pl.when(i + 2 < per_w)
      def _():
        idesc(i + 2, slot).start()

      wdesc(i, slot).start()
      return carry

    lax.fori_loop(0, per_w, body, 0)
    wdesc(per_w - 2, per_w % 2).wait()
    wdesc(per_w - 1, (per_w - 1) % 2).wait()

  return gather


def kernel(x, x_0):
  B, H = x.shape
  V = x_0.shape[0]
  table = _fmt_fn(V)(x_0.T)
  y = _gather_fn(B, H)(table, x.T)
  return jnp.transpose(y, (2, 0, 1))
